# baseline (speedup 1.0000x reference)
import functools

import jax
import jax.numpy as jnp
from jax import lax
from jax.experimental import pallas as pl
from jax.experimental.pallas import tpu as pltpu

BATCH = 1024
EMBED_DIM = 32
IMG_SHAPE = (3, 775, 120)
OUT_DIM = 3 * 775 * 120

_DEPTH = 32


def _gather_body(labels_ref, table_ref, x_ref, sem):
    def _cp(i):
        row = labels_ref[i]
        return pltpu.make_async_copy(
            table_ref.at[pl.ds(row, 1), :],
            x_ref.at[pl.ds(i, 1), :],
            sem.at[lax.rem(i, _DEPTH)],
        )

    def _step(i, c):
        _cp(i).start()

        @pl.when(i >= _DEPTH)
        def _():
            _cp(i - _DEPTH).wait()

        return c

    lax.fori_loop(0, BATCH, _step, 0)

    def _drain(i, c):
        _cp(BATCH - _DEPTH + i).wait()
        return c

    lax.fori_loop(0, _DEPTH, _drain, 0)


_H_T = 32
_N_HT = (775 + _H_T - 1) // _H_T
_W_BLK = _H_T * 120


def _mm_body(x_ref, w_ref, b_ref, o_ref):
    x = x_ref[...]
    for h in range(_H_T):
        w = w_ref[0, pl.ds(h * 120, 120), :]
        acc = lax.dot_general(
            x, w, (((1,), (1,)), ((), ())),
            preferred_element_type=jnp.float32,
        )
        o_ref[:, 0, h, :] = acc + b_ref[0, 0, h, :]


@jax.jit
def kernel(labels, embed_table, W, b):
    x = pl.pallas_call(
        _gather_body,
        in_specs=[
            pl.BlockSpec(memory_space=pltpu.SMEM),
            pl.BlockSpec(memory_space=pl.ANY),
        ],
        out_specs=pl.BlockSpec(memory_space=pltpu.VMEM),
        out_shape=jax.ShapeDtypeStruct((BATCH, EMBED_DIM), jnp.float32),
        scratch_shapes=[pltpu.SemaphoreType.DMA((_DEPTH,))],
    )(labels, embed_table)

    b4 = b.reshape(1, *IMG_SHAPE)
    W3 = W.reshape(3, 775 * 120, EMBED_DIM)
    y = pl.pallas_call(
        _mm_body,
        grid=(3, _N_HT),
        in_specs=[
            pl.BlockSpec((BATCH, EMBED_DIM), lambda c, t: (0, 0)),
            pl.BlockSpec((1, _W_BLK, EMBED_DIM), lambda c, t: (c, t, 0)),
            pl.BlockSpec((1, 1, _H_T, 120), lambda c, t: (0, c, t, 0)),
        ],
        out_specs=pl.BlockSpec(
            (BATCH, 1, _H_T, 120), lambda c, t: (0, c, t, 0)
        ),
        out_shape=jax.ShapeDtypeStruct((BATCH,) + IMG_SHAPE, jnp.float32),
        compiler_params=pltpu.CompilerParams(
            dimension_semantics=("parallel", "parallel"),
        ),
    )(x, W3, b4)
    return y


# speedup vs baseline: 5.8215x; 5.8215x over previous
import jax
import jax.numpy as jnp
from jax import lax
from jax.experimental import pallas as pl
from jax.experimental.pallas import tpu as pltpu

BATCH = 1024
EMBED_DIM = 32
IMG_SHAPE = (3, 775, 120)
OUT_DIM = 3 * 775 * 120

_DEPTH = 32


def _gather_body(labels_ref, oh_ref, tableT_ref, x_ref, panels, sem):
    def _cp(i):
        base = pl.multiple_of((labels_ref[i] // 128) * 128, 128)
        return pltpu.make_async_copy(
            tableT_ref.at[:, pl.ds(base, 128)],
            panels.at[i],
            sem.at[lax.rem(i, _DEPTH)],
        )

    def _step(i, c):
        _cp(i).start()

        @pl.when(i >= _DEPTH)
        def _():
            _cp(i - _DEPTH).wait()

        return c

    lax.fori_loop(0, BATCH, _step, 0)

    def _drain(i, c):
        _cp(BATCH - _DEPTH + i).wait()
        return c

    lax.fori_loop(0, _DEPTH, _drain, 0)

    x = jnp.sum(panels[...] * oh_ref[...], axis=2)
    x_ref[...] = jnp.concatenate(
        [x, jnp.ones((BATCH, 1), jnp.float32)], axis=1
    )


_N_T = 3840
_GRID = (OUT_DIM + _N_T - 1) // _N_T


def _mm_body(x_ref, wT_ref, b_ref, o_ref):
    waug = jnp.concatenate([wT_ref[...], b_ref[...]], axis=0)
    o_ref[...] = lax.dot_general(
        waug, x_ref[...], (((0,), (1,)), ((), ())),
        preferred_element_type=jnp.float32,
    )


@jax.jit
def kernel(labels, embed_table, W, b):
    tableT = embed_table.T
    WT = W.T
    oh = jax.nn.one_hot(labels % 128, 128, dtype=jnp.float32)
    x = pl.pallas_call(
        _gather_body,
        in_specs=[
            pl.BlockSpec(memory_space=pltpu.SMEM),
            pl.BlockSpec(memory_space=pltpu.VMEM),
            pl.BlockSpec(memory_space=pl.ANY),
        ],
        out_specs=pl.BlockSpec(memory_space=pltpu.VMEM),
        out_shape=jax.ShapeDtypeStruct((BATCH, EMBED_DIM + 1), jnp.float32),
        scratch_shapes=[
            pltpu.VMEM((BATCH, EMBED_DIM, 128), jnp.float32),
            pltpu.SemaphoreType.DMA((_DEPTH,)),
        ],
    )(labels, oh.reshape(BATCH, 1, 128), tableT)

    b2 = b.reshape(1, OUT_DIM)
    yT = pl.pallas_call(
        _mm_body,
        grid=(_GRID,),
        in_specs=[
            pl.BlockSpec((BATCH, EMBED_DIM + 1), lambda n: (0, 0)),
            pl.BlockSpec((EMBED_DIM, _N_T), lambda n: (0, n)),
            pl.BlockSpec((1, _N_T), lambda n: (0, n)),
        ],
        out_specs=pl.BlockSpec((_N_T, BATCH), lambda n: (n, 0)),
        out_shape=jax.ShapeDtypeStruct((OUT_DIM, BATCH), jnp.float32),
        compiler_params=pltpu.CompilerParams(
            dimension_semantics=("parallel",),
        ),
    )(x, WT, b2)
    y = yT.reshape(*IMG_SHAPE, BATCH).transpose(3, 0, 1, 2)
    return y


# speedup vs baseline: 6.2347x; 1.0710x over previous
import jax
import jax.numpy as jnp
from jax import lax
from jax.experimental import pallas as pl
from jax.experimental.pallas import tpu as pltpu

BATCH = 1024
EMBED_DIM = 32
IMG_SHAPE = (3, 775, 120)
OUT_DIM = 3 * 775 * 120

_DEPTH = 32


_UNROLL = 8


def _gather_body(labels_ref, oh_ref, tableT_ref, x_ref, panels, sem):
    def _issue(j, c):
        for u in range(_UNROLL):
            i = j * _UNROLL + u
            base = pl.multiple_of((labels_ref[i] // 128) * 128, 128)
            pltpu.make_async_copy(
                tableT_ref.at[:, pl.ds(base, 128)],
                panels.at[i],
                sem.at[0],
            ).start()
        return c

    lax.fori_loop(0, BATCH // _UNROLL, _issue, 0)

    def _drain(j, c):
        for _ in range(_UNROLL):
            pltpu.make_async_copy(
                tableT_ref.at[:, pl.ds(0, 128)], panels.at[0], sem.at[0]
            ).wait()
        return c

    lax.fori_loop(0, BATCH // _UNROLL, _drain, 0)

    x = jnp.sum(panels[...] * oh_ref[...], axis=2)
    x_ref[...] = jnp.concatenate(
        [x, jnp.ones((BATCH, 1), jnp.float32)], axis=1
    )


_N_T = 4096
_GRID = (OUT_DIM + _N_T - 1) // _N_T


def _mm_body(x_ref, wT_ref, b_ref, o_ref):
    waug = jnp.concatenate([wT_ref[...], b_ref[...][None, :]], axis=0)
    o_ref[...] = lax.dot_general(
        waug, x_ref[...], (((0,), (1,)), ((), ())),
        preferred_element_type=jnp.float32,
    )


@jax.jit
def kernel(labels, embed_table, W, b):
    tableT = embed_table.T
    WT = W.T
    oh = jax.nn.one_hot(labels % 128, 128, dtype=jnp.float32)
    x = pl.pallas_call(
        _gather_body,
        in_specs=[
            pl.BlockSpec(memory_space=pltpu.SMEM),
            pl.BlockSpec(memory_space=pltpu.VMEM),
            pl.BlockSpec(memory_space=pl.ANY),
        ],
        out_specs=pl.BlockSpec(memory_space=pltpu.VMEM),
        out_shape=jax.ShapeDtypeStruct((BATCH, EMBED_DIM + 1), jnp.float32),
        scratch_shapes=[
            pltpu.VMEM((BATCH, EMBED_DIM, 128), jnp.float32),
            pltpu.SemaphoreType.DMA((_DEPTH,)),
        ],
    )(labels, oh.reshape(BATCH, 1, 128), tableT)

    yT = pl.pallas_call(
        _mm_body,
        grid=(_GRID,),
        in_specs=[
            pl.BlockSpec((BATCH, EMBED_DIM + 1), lambda n: (0, 0)),
            pl.BlockSpec((EMBED_DIM, _N_T), lambda n: (0, n)),
            pl.BlockSpec((_N_T,), lambda n: (n,)),
        ],
        out_specs=pl.BlockSpec((_N_T, BATCH), lambda n: (n, 0)),
        out_shape=jax.ShapeDtypeStruct((OUT_DIM, BATCH), jnp.float32),
        compiler_params=pltpu.CompilerParams(
            dimension_semantics=("parallel",),
        ),
    )(x, WT, b)
    y = yT.reshape(*IMG_SHAPE, BATCH).transpose(3, 0, 1, 2)
    return y
